# baseline (speedup 1.0000x reference)
import functools

import jax
import jax.numpy as jnp
from jax import lax
from jax.experimental import pallas as pl
from jax.experimental.pallas import tpu as pltpu
from jax.experimental.pallas import tpu_sc as plsc

B = 4096
P = 2
L_SENT = 200
D = 64

D_SC = 32
D_TC = D - D_SC

_info = plsc.get_sparse_core_info()
_NC, _NS, _NL = _info.num_cores, _info.num_subcores, _info.num_lanes
_NW = _NC * _NS
ROWS = B * P
R_PER_W = ROWS // _NW
WAVE = 8
N_WAVES = R_PER_W // WAVE
BT = 128


def _sc_gather(table_t, idx_flat):
    mesh = plsc.VectorSubcoreMesh(core_axis_name="c", subcore_axis_name="s")

    @functools.partial(
        pl.kernel,
        mesh=mesh,
        out_type=jax.ShapeDtypeStruct((ROWS, D_SC), jnp.float32),
        compiler_params=pltpu.CompilerParams(needs_layout_passes=False),
        scratch_types=[
            pltpu.VMEM((R_PER_W,), jnp.int32),
            pltpu.VMEM((2, WAVE, D_SC, BT), jnp.float32),
            pltpu.VMEM((R_PER_W, D_SC), jnp.float32),
            pltpu.SemaphoreType.DMA,
            pltpu.SemaphoreType.DMA,
        ],
    )
    def k(table_hbm, idx_hbm, out_hbm, raw_v, slab_v, rows_v, sem0, sem1):
        wid = lax.axis_index("s") * _NC + lax.axis_index("c")
        base = wid * R_PER_W
        pltpu.sync_copy(idx_hbm.at[pl.ds(base, R_PER_W)], raw_v)
        lane = lax.iota(jnp.int32, _NL)
        zero = jnp.zeros((_NL,), jnp.int32)
        sems = (sem0, sem1)

        def fire(w):
            buf = w % 2
            for k_ in range(WAVE):
                r = w * WAVE + k_
                vec = raw_v[pl.ds((r // _NL) * _NL, _NL)]
                pos = jnp.sum(jnp.where(lane == (r % _NL), vec, zero))
                b_abs = (base + r) >> 1
                pltpu.make_async_copy(
                    table_hbm.at[pl.ds(pos, 1), pl.ds(0, D_SC),
                                 pl.ds((b_abs // BT) * BT, BT)],
                    slab_v.at[buf].at[pl.ds(k_, 1)],
                    sems[buf],
                ).start()

        def drain(w):
            buf = w % 2
            pltpu.make_async_copy(
                table_hbm.at[pl.ds(0, 1), pl.ds(0, D_SC), pl.ds(0, BT)],
                slab_v.at[buf].at[pl.ds(0, 1)],
                sems[buf],
            ).wait()

        def select(w):
            buf = w % 2
            for k_ in range(WAVE):
                r = w * WAVE + k_
                b_in = ((base + r) >> 1) % BT
                col = jnp.full((_NL,), b_in, dtype=jnp.int32)
                kvec = jnp.full((_NL,), k_, dtype=jnp.int32)
                for d0 in range(0, D_SC, _NL):
                    dvec = lane + d0
                    val = plsc.load_gather(
                        slab_v.at[buf], [kvec, dvec, col])
                    rows_v[r, pl.ds(d0, _NL)] = val

        fire(0)
        for w in range(N_WAVES):
            if w + 1 < N_WAVES:
                fire(w + 1)
            for k_ in range(WAVE - 1):
                drain(w)
            drain(w)
            select(w)
        pltpu.sync_copy(rows_v, out_hbm.at[pl.ds(base, R_PER_W)])

    return k(table_t, idx_flat)


B_BLK = 4096


def _tc_body(idx_ref, tab_ref, out_ref):
    i = pl.program_id(0)

    @pl.when(i == 0)
    def _init():
        out_ref[...] = jnp.zeros_like(out_ref)

    tab = tab_ref[0]
    lvals = idx_ref[...]
    for p in range(P):
        sel = lvals[p, :][None, :] == i
        out_ref[p] = jnp.where(sel, tab, out_ref[p])


def _tc_gather(table_t, idx_t):
    return pl.pallas_call(
        _tc_body,
        grid=(L_SENT,),
        in_specs=[
            pl.BlockSpec((P, B_BLK), lambda i: (0, 0)),
            pl.BlockSpec((1, D_TC, B_BLK), lambda i: (i, 1, 0)),
        ],
        out_specs=pl.BlockSpec((P, D_TC, B_BLK), lambda i: (0, 0, 0)),
        out_shape=jax.ShapeDtypeStruct((P, D_TC, B), jnp.float32),
    )(idx_t, table_t)


def kernel(sentence_matrix, entity_pos_index):
    table_t = jnp.transpose(sentence_matrix, (1, 2, 0))
    idx32 = entity_pos_index.astype(jnp.int32)
    sc_rows = _sc_gather(table_t, idx32.reshape(ROWS))
    tc_out = _tc_gather(table_t, idx32.T)
    lo = sc_rows.reshape(B, P, D_SC)
    hi = jnp.transpose(tc_out, (2, 0, 1))
    return jnp.concatenate([lo, hi], axis=2)


# speedup vs baseline: 1.4727x; 1.4727x over previous
import functools

import jax
import jax.numpy as jnp
from jax import lax
from jax.experimental import pallas as pl
from jax.experimental.pallas import tpu as pltpu
from jax.experimental.pallas import tpu_sc as plsc

B = 4096
P = 2
L_SENT = 200
D = 64

D_SC = 32
D_TC = D - D_SC

_info = plsc.get_sparse_core_info()
_NC, _NS, _NL = _info.num_cores, _info.num_subcores, _info.num_lanes
_NW = _NC * _NS
ROWS = B * P
R_PER_W = ROWS // _NW
WAVE = 8
N_WAVES = R_PER_W // WAVE
BT = 128


def _sc_gather(table_t, idx_flat):
    mesh = plsc.VectorSubcoreMesh(core_axis_name="c", subcore_axis_name="s")

    @functools.partial(
        pl.kernel,
        mesh=mesh,
        out_type=jax.ShapeDtypeStruct((ROWS, D_SC), jnp.float32),
        compiler_params=pltpu.CompilerParams(needs_layout_passes=False),
        scratch_types=[
            pltpu.VMEM((R_PER_W,), jnp.int32),
            pltpu.VMEM((2, WAVE, D_SC, BT), jnp.float32),
            pltpu.VMEM((R_PER_W, D_SC), jnp.float32),
            pltpu.SemaphoreType.DMA,
            pltpu.SemaphoreType.DMA,
        ],
    )
    def k(table_hbm, idx_hbm, out_hbm, raw_v, slab_v, rows_v, sem0, sem1):
        wid = lax.axis_index("s") * _NC + lax.axis_index("c")
        base = wid * R_PER_W
        pltpu.sync_copy(idx_hbm.at[pl.ds(base, R_PER_W)], raw_v)
        lane = lax.iota(jnp.int32, _NL)
        zero = jnp.zeros((_NL,), jnp.int32)
        sems = (sem0, sem1)

        def fire(w):
            buf = w % 2
            for k_ in range(WAVE):
                r = w * WAVE + k_
                vec = raw_v[pl.ds((r // _NL) * _NL, _NL)]
                pos = jnp.sum(jnp.where(lane == (r % _NL), vec, zero))
                b_abs = (base + r) >> 1
                pltpu.make_async_copy(
                    table_hbm.at[pl.ds(pos, 1), pl.ds(0, D_SC),
                                 pl.ds((b_abs // BT) * BT, BT)],
                    slab_v.at[buf].at[pl.ds(k_, 1)],
                    sems[buf],
                ).start()

        def drain(w):
            buf = w % 2
            pltpu.make_async_copy(
                table_hbm.at[pl.ds(0, 1), pl.ds(0, D_SC), pl.ds(0, BT)],
                slab_v.at[buf].at[pl.ds(0, 1)],
                sems[buf],
            ).wait()

        def select(w):
            buf = w % 2
            for k_ in range(WAVE):
                r = w * WAVE + k_
                b_in = ((base + r) >> 1) % BT
                col = jnp.full((_NL,), b_in, dtype=jnp.int32)
                kvec = jnp.full((_NL,), k_, dtype=jnp.int32)
                for d0 in range(0, D_SC, _NL):
                    dvec = lane + d0
                    val = plsc.load_gather(
                        slab_v.at[buf], [kvec, dvec, col])
                    rows_v[r, pl.ds(d0, _NL)] = val

        fire(0)
        for w in range(N_WAVES):
            if w + 1 < N_WAVES:
                fire(w + 1)
            for k_ in range(WAVE - 1):
                drain(w)
            drain(w)
            select(w)
        pltpu.sync_copy(rows_v, out_hbm.at[pl.ds(base, R_PER_W)])

    return k(table_t, idx_flat)


B_BLK = 4096
CHUNK_L = 8
N_LSTEP = L_SENT // CHUNK_L


def _tc_body(idx_ref, tab_ref, out_ref):
    i = pl.program_id(0)

    @pl.when(i == 0)
    def _init():
        out_ref[...] = jnp.zeros_like(out_ref)

    lvals = idx_ref[...]
    for q in range(CHUNK_L):
        tab = tab_ref[q]
        l_cur = i * CHUNK_L + q
        for p in range(P):
            sel = lvals[p, :][None, :] == l_cur
            out_ref[p] = jnp.where(sel, tab, out_ref[p])


def _tc_gather(table_t, idx_t):
    return pl.pallas_call(
        _tc_body,
        grid=(N_LSTEP,),
        in_specs=[
            pl.BlockSpec((P, B_BLK), lambda i: (0, 0)),
            pl.BlockSpec((CHUNK_L, D_TC, B_BLK), lambda i: (i, 1, 0)),
        ],
        out_specs=pl.BlockSpec((P, D_TC, B_BLK), lambda i: (0, 0, 0)),
        out_shape=jax.ShapeDtypeStruct((P, D_TC, B), jnp.float32),
    )(idx_t, table_t)


def kernel(sentence_matrix, entity_pos_index):
    table_t = jnp.transpose(sentence_matrix, (1, 2, 0))
    idx32 = entity_pos_index.astype(jnp.int32)
    sc_rows = _sc_gather(table_t, idx32.reshape(ROWS))
    tc_out = _tc_gather(table_t, idx32.T)
    lo = sc_rows.reshape(B, P, D_SC)
    hi = jnp.transpose(tc_out, (2, 0, 1))
    return jnp.concatenate([lo, hi], axis=2)


# speedup vs baseline: 1.4965x; 1.0161x over previous
import functools

import jax
import jax.numpy as jnp
from jax import lax
from jax.experimental import pallas as pl
from jax.experimental.pallas import tpu as pltpu
from jax.experimental.pallas import tpu_sc as plsc

B = 4096
P = 2
L_SENT = 200
D = 64

D_SC = 32
D_TC = D - D_SC

_info = plsc.get_sparse_core_info()
_NC, _NS, _NL = _info.num_cores, _info.num_subcores, _info.num_lanes
_NW = _NC * _NS
ROWS = B * P
R_PER_W = ROWS // _NW
WAVE = 8
N_WAVES = R_PER_W // WAVE
BT = 128


def _sc_gather(table_t, idx_flat):
    mesh = plsc.VectorSubcoreMesh(core_axis_name="c", subcore_axis_name="s")

    @functools.partial(
        pl.kernel,
        mesh=mesh,
        out_type=jax.ShapeDtypeStruct((ROWS, D_SC), jnp.float32),
        compiler_params=pltpu.CompilerParams(needs_layout_passes=False),
        scratch_types=[
            pltpu.VMEM((R_PER_W,), jnp.int32),
            pltpu.VMEM((2, WAVE, D_SC, BT), jnp.float32),
            pltpu.VMEM((R_PER_W, D_SC), jnp.float32),
            pltpu.SemaphoreType.DMA,
            pltpu.SemaphoreType.DMA,
        ],
    )
    def k(table_hbm, idx_hbm, out_hbm, raw_v, slab_v, rows_v, sem0, sem1):
        wid = lax.axis_index("s") * _NC + lax.axis_index("c")
        base = wid * R_PER_W
        pltpu.sync_copy(idx_hbm.at[pl.ds(base, R_PER_W)], raw_v)
        lane = lax.iota(jnp.int32, _NL)
        zero = jnp.zeros((_NL,), jnp.int32)
        sems = (sem0, sem1)

        def fire(w):
            buf = w % 2
            for k_ in range(WAVE):
                r = w * WAVE + k_
                vec = raw_v[pl.ds((r // _NL) * _NL, _NL)]
                pos = jnp.sum(jnp.where(lane == (r % _NL), vec, zero))
                b_abs = (base + r) >> 1
                pltpu.make_async_copy(
                    table_hbm.at[pl.ds(pos, 1), pl.ds(0, D_SC),
                                 pl.ds((b_abs // BT) * BT, BT)],
                    slab_v.at[buf].at[pl.ds(k_, 1)],
                    sems[buf],
                ).start()

        def drain(w):
            buf = w % 2
            pltpu.make_async_copy(
                table_hbm.at[pl.ds(0, 1), pl.ds(0, D_SC), pl.ds(0, BT)],
                slab_v.at[buf].at[pl.ds(0, 1)],
                sems[buf],
            ).wait()

        def select(w):
            buf = w % 2
            for k_ in range(WAVE):
                r = w * WAVE + k_
                b_in = ((base + r) >> 1) % BT
                col = jnp.full((_NL,), b_in, dtype=jnp.int32)
                kvec = jnp.full((_NL,), k_, dtype=jnp.int32)
                for d0 in range(0, D_SC, _NL):
                    dvec = lane + d0
                    val = plsc.load_gather(
                        slab_v.at[buf], [kvec, dvec, col])
                    rows_v[r, pl.ds(d0, _NL)] = val

        fire(0)
        for w in range(N_WAVES):
            if w + 1 < N_WAVES:
                fire(w + 1)
            for k_ in range(WAVE - 1):
                drain(w)
            drain(w)
            select(w)
        pltpu.sync_copy(rows_v, out_hbm.at[pl.ds(base, R_PER_W)])

    return k(table_t, idx_flat)


B_BLK = 4096
CHUNK_L = 25
N_LSTEP = L_SENT // CHUNK_L


def _tc_body(idx_ref, tab_ref, out_ref):
    i = pl.program_id(0)

    @pl.when(i == 0)
    def _init():
        out_ref[...] = jnp.zeros_like(out_ref)

    lvals = idx_ref[...]
    for q in range(CHUNK_L):
        tab = tab_ref[q]
        l_cur = i * CHUNK_L + q
        for p in range(P):
            sel = lvals[p, :][None, :] == l_cur
            out_ref[p] = jnp.where(sel, tab, out_ref[p])


def _tc_gather(table_t, idx_t):
    return pl.pallas_call(
        _tc_body,
        grid=(N_LSTEP,),
        in_specs=[
            pl.BlockSpec((P, B_BLK), lambda i: (0, 0)),
            pl.BlockSpec((CHUNK_L, D_TC, B_BLK), lambda i: (i, 1, 0)),
        ],
        out_specs=pl.BlockSpec((P, D_TC, B_BLK), lambda i: (0, 0, 0)),
        out_shape=jax.ShapeDtypeStruct((P, D_TC, B), jnp.float32),
    )(idx_t, table_t)


def kernel(sentence_matrix, entity_pos_index):
    table_t = jnp.transpose(sentence_matrix, (1, 2, 0))
    idx32 = entity_pos_index.astype(jnp.int32)
    sc_rows = _sc_gather(table_t, idx32.reshape(ROWS))
    tc_out = _tc_gather(table_t, idx32.T)
    lo = sc_rows.reshape(B, P, D_SC)
    hi = jnp.transpose(tc_out, (2, 0, 1))
    return jnp.concatenate([lo, hi], axis=2)


# speedup vs baseline: 1.5244x; 1.0187x over previous
import functools

import jax
import jax.numpy as jnp
from jax import lax
from jax.experimental import pallas as pl
from jax.experimental.pallas import tpu as pltpu
from jax.experimental.pallas import tpu_sc as plsc

B = 4096
P = 2
L_SENT = 200
D = 64

D_SC = 24
D_TC = D - D_SC

_info = plsc.get_sparse_core_info()
_NC, _NS, _NL = _info.num_cores, _info.num_subcores, _info.num_lanes
_NW = _NC * _NS
ROWS = B * P
R_PER_W = ROWS // _NW
WAVE = 8
N_WAVES = R_PER_W // WAVE
BT = 128


def _sc_gather(table_t, idx_flat):
    mesh = plsc.VectorSubcoreMesh(core_axis_name="c", subcore_axis_name="s")

    @functools.partial(
        pl.kernel,
        mesh=mesh,
        out_type=jax.ShapeDtypeStruct((ROWS, D_SC), jnp.float32),
        compiler_params=pltpu.CompilerParams(needs_layout_passes=False),
        scratch_types=[
            pltpu.VMEM((R_PER_W,), jnp.int32),
            pltpu.VMEM((2, WAVE, D_SC, BT), jnp.float32),
            pltpu.VMEM((R_PER_W, D_SC), jnp.float32),
            pltpu.SemaphoreType.DMA,
            pltpu.SemaphoreType.DMA,
        ],
    )
    def k(table_hbm, idx_hbm, out_hbm, raw_v, slab_v, rows_v, sem0, sem1):
        wid = lax.axis_index("s") * _NC + lax.axis_index("c")
        base = wid * R_PER_W
        pltpu.sync_copy(idx_hbm.at[pl.ds(base, R_PER_W)], raw_v)
        lane = lax.iota(jnp.int32, _NL)
        zero = jnp.zeros((_NL,), jnp.int32)
        sems = (sem0, sem1)

        def fire(w):
            buf = w % 2
            for k_ in range(WAVE):
                r = w * WAVE + k_
                vec = raw_v[pl.ds((r // _NL) * _NL, _NL)]
                pos = jnp.sum(jnp.where(lane == (r % _NL), vec, zero))
                b_abs = (base + r) >> 1
                pltpu.make_async_copy(
                    table_hbm.at[pl.ds(pos, 1), pl.ds(0, D_SC),
                                 pl.ds((b_abs // BT) * BT, BT)],
                    slab_v.at[buf].at[pl.ds(k_, 1)],
                    sems[buf],
                ).start()

        def drain(w):
            buf = w % 2
            pltpu.make_async_copy(
                table_hbm.at[pl.ds(0, 1), pl.ds(0, D_SC), pl.ds(0, BT)],
                slab_v.at[buf].at[pl.ds(0, 1)],
                sems[buf],
            ).wait()

        def select(w):
            buf = w % 2
            for k_ in range(WAVE):
                r = w * WAVE + k_
                b_in = ((base + r) >> 1) % BT
                col = jnp.full((_NL,), b_in, dtype=jnp.int32)
                kvec = jnp.full((_NL,), k_, dtype=jnp.int32)
                starts = range(0, D_SC, _NL) if D_SC % _NL == 0 else (
                    [0, D_SC - _NL])
                for d0 in starts:
                    dvec = lane + d0
                    val = plsc.load_gather(
                        slab_v.at[buf], [kvec, dvec, col])
                    rows_v[r, pl.ds(d0, _NL)] = val

        fire(0)
        for w in range(N_WAVES):
            if w + 1 < N_WAVES:
                fire(w + 1)
            for k_ in range(WAVE - 1):
                drain(w)
            drain(w)
            select(w)
        pltpu.sync_copy(rows_v, out_hbm.at[pl.ds(base, R_PER_W)])

    return k(table_t, idx_flat)


B_BLK = 4096
CHUNK_L = 25
N_LSTEP = L_SENT // CHUNK_L
D_BLK = 8
N_DSTEP = D_TC // D_BLK
D_SKIP = D_SC // D_BLK


def _tc_body(idx_ref, tab_ref, out_ref):
    i = pl.program_id(1)

    @pl.when(i == 0)
    def _init():
        out_ref[...] = jnp.zeros_like(out_ref)

    lvals = idx_ref[...]
    for q in range(CHUNK_L):
        tab = tab_ref[q]
        l_cur = i * CHUNK_L + q
        for p in range(P):
            sel = lvals[p, :][None, :] == l_cur
            out_ref[p] = jnp.where(sel, tab, out_ref[p])


def _tc_gather(table_t, idx_t):
    return pl.pallas_call(
        _tc_body,
        grid=(N_DSTEP, N_LSTEP),
        in_specs=[
            pl.BlockSpec((P, B_BLK), lambda j, i: (0, 0)),
            pl.BlockSpec((CHUNK_L, D_BLK, B_BLK),
                         lambda j, i: (i, D_SKIP + j, 0)),
        ],
        out_specs=pl.BlockSpec((P, D_BLK, B_BLK), lambda j, i: (0, j, 0)),
        out_shape=jax.ShapeDtypeStruct((P, D_TC, B), jnp.float32),
    )(idx_t, table_t)


def kernel(sentence_matrix, entity_pos_index):
    table_t = jnp.transpose(sentence_matrix, (1, 2, 0))
    idx32 = entity_pos_index.astype(jnp.int32)
    sc_rows = _sc_gather(table_t, idx32.reshape(ROWS))
    tc_out = _tc_gather(table_t, idx32.T)
    lo = sc_rows.reshape(B, P, D_SC)
    hi = jnp.transpose(tc_out, (2, 0, 1))
    return jnp.concatenate([lo, hi], axis=2)


# speedup vs baseline: 1.5876x; 1.0414x over previous
import functools

import jax
import jax.numpy as jnp
from jax import lax
from jax.experimental import pallas as pl
from jax.experimental.pallas import tpu as pltpu
from jax.experimental.pallas import tpu_sc as plsc

B = 4096
P = 2
L_SENT = 200
D = 64

D_SC = 24
D_TC = D - D_SC

_info = plsc.get_sparse_core_info()
_NC, _NS, _NL = _info.num_cores, _info.num_subcores, _info.num_lanes
_NW = _NC * _NS
ROWS = B * P
R_PER_W = ROWS // _NW
WAVE = 8
N_WAVES = R_PER_W // WAVE
BT = 128


def _sc_gather(table_t, idx_flat):
    mesh = plsc.VectorSubcoreMesh(core_axis_name="c", subcore_axis_name="s")

    @functools.partial(
        pl.kernel,
        mesh=mesh,
        out_type=jax.ShapeDtypeStruct((ROWS, D_SC), jnp.float32),
        compiler_params=pltpu.CompilerParams(needs_layout_passes=False),
        scratch_types=[
            pltpu.VMEM((R_PER_W,), jnp.int32),
            pltpu.VMEM((2, WAVE, D_SC, BT), jnp.float32),
            pltpu.VMEM((R_PER_W, D_SC), jnp.float32),
            pltpu.SemaphoreType.DMA,
            pltpu.SemaphoreType.DMA,
        ],
    )
    def k(table_hbm, idx_hbm, out_hbm, raw_v, slab_v, rows_v, sem0, sem1):
        wid = lax.axis_index("s") * _NC + lax.axis_index("c")
        base = wid * R_PER_W
        pltpu.sync_copy(idx_hbm.at[pl.ds(base, R_PER_W)], raw_v)
        lane = lax.iota(jnp.int32, _NL)
        zero = jnp.zeros((_NL,), jnp.int32)
        sems = (sem0, sem1)

        def fire(w):
            buf = w % 2
            for k_ in range(WAVE):
                r = w * WAVE + k_
                vec = raw_v[pl.ds((r // _NL) * _NL, _NL)]
                pos = jnp.sum(jnp.where(lane == (r % _NL), vec, zero))
                b_abs = (base + r) >> 1
                pltpu.make_async_copy(
                    table_hbm.at[pl.ds(pos, 1), pl.ds(0, D_SC),
                                 pl.ds((b_abs // BT) * BT, BT)],
                    slab_v.at[buf].at[pl.ds(k_, 1)],
                    sems[buf],
                ).start()

        def drain(w):
            buf = w % 2
            pltpu.make_async_copy(
                table_hbm.at[pl.ds(0, 1), pl.ds(0, D_SC), pl.ds(0, BT)],
                slab_v.at[buf].at[pl.ds(0, 1)],
                sems[buf],
            ).wait()

        def select(w):
            buf = w % 2
            for k_ in range(WAVE):
                r = w * WAVE + k_
                b_in = ((base + r) >> 1) % BT
                col = jnp.full((_NL,), b_in, dtype=jnp.int32)
                kvec = jnp.full((_NL,), k_, dtype=jnp.int32)
                starts = range(0, D_SC, _NL) if D_SC % _NL == 0 else (
                    [0, D_SC - _NL])
                for d0 in starts:
                    dvec = lane + d0
                    val = plsc.load_gather(
                        slab_v.at[buf], [kvec, dvec, col])
                    rows_v[r, pl.ds(d0, _NL)] = val

        fire(0)
        for w in range(N_WAVES):
            if w + 1 < N_WAVES:
                fire(w + 1)
            for k_ in range(WAVE - 1):
                drain(w)
            drain(w)
            select(w)
        pltpu.sync_copy(rows_v, out_hbm.at[pl.ds(base, R_PER_W)])

    return k(table_t, idx_flat)


B_BLK = 4096
CHUNK_L = 100
N_LSTEP = L_SENT // CHUNK_L
D_BLK = 8
N_DSTEP = D_TC // D_BLK
D_SKIP = D_SC // D_BLK


def _tc_body(idx_ref, tab_ref, out_ref):
    i = pl.program_id(1)

    @pl.when(i == 0)
    def _init():
        out_ref[...] = jnp.zeros_like(out_ref)

    lvals = idx_ref[...]
    for q in range(CHUNK_L):
        tab = tab_ref[q]
        l_cur = i * CHUNK_L + q
        for p in range(P):
            sel = lvals[p, :][None, :] == l_cur
            out_ref[p] = jnp.where(sel, tab, out_ref[p])


def _tc_gather(table_t, idx_t):
    return pl.pallas_call(
        _tc_body,
        grid=(N_DSTEP, N_LSTEP),
        in_specs=[
            pl.BlockSpec((P, B_BLK), lambda j, i: (0, 0)),
            pl.BlockSpec((CHUNK_L, D_BLK, B_BLK),
                         lambda j, i: (i, D_SKIP + j, 0)),
        ],
        out_specs=pl.BlockSpec((P, D_BLK, B_BLK), lambda j, i: (0, j, 0)),
        out_shape=jax.ShapeDtypeStruct((P, D_TC, B), jnp.float32),
    )(idx_t, table_t)


def kernel(sentence_matrix, entity_pos_index):
    table_t = jnp.transpose(sentence_matrix, (1, 2, 0))
    idx32 = entity_pos_index.astype(jnp.int32)
    sc_rows = _sc_gather(table_t, idx32.reshape(ROWS))
    tc_out = _tc_gather(table_t, idx32.T)
    lo = sc_rows.reshape(B, P, D_SC)
    hi = jnp.transpose(tc_out, (2, 0, 1))
    return jnp.concatenate([lo, hi], axis=2)


# speedup vs baseline: 1.6735x; 1.0541x over previous
import functools

import jax
import jax.numpy as jnp
from jax import lax
from jax.experimental import pallas as pl
from jax.experimental.pallas import tpu as pltpu
from jax.experimental.pallas import tpu_sc as plsc

B = 4096
P = 2
L_SENT = 200
D = 64

D_SC = 24
D_TC = D - D_SC

_info = plsc.get_sparse_core_info()
_NC, _NS, _NL = _info.num_cores, _info.num_subcores, _info.num_lanes
_NW = _NC * _NS
ROWS = B * P
R_PER_W = ROWS // _NW
WAVE = 8
N_WAVES = R_PER_W // WAVE
BT = 128


def _sc_gather(table_t, idx_flat):
    mesh = plsc.VectorSubcoreMesh(core_axis_name="c", subcore_axis_name="s")

    @functools.partial(
        pl.kernel,
        mesh=mesh,
        out_type=jax.ShapeDtypeStruct((P, D_SC, B), jnp.float32),
        compiler_params=pltpu.CompilerParams(needs_layout_passes=False),
        scratch_types=[
            pltpu.VMEM((R_PER_W,), jnp.int32),
            pltpu.VMEM((2, WAVE, D_SC, BT), jnp.float32),
            pltpu.VMEM((P, D_SC, BT), jnp.float32),
            pltpu.SemaphoreType.DMA,
            pltpu.SemaphoreType.DMA,
        ],
    )
    def k(table_hbm, idx_hbm, out_hbm, raw_v, slab_v, rows_v, sem0, sem1):
        wid = lax.axis_index("s") * _NC + lax.axis_index("c")
        base = wid * R_PER_W
        pltpu.sync_copy(idx_hbm.at[pl.ds(base, R_PER_W)], raw_v)
        lane = lax.iota(jnp.int32, _NL)
        zero = jnp.zeros((_NL,), jnp.int32)
        sems = (sem0, sem1)

        def fire(w):
            buf = w % 2
            for k_ in range(WAVE):
                r = w * WAVE + k_
                vec = raw_v[pl.ds((r // _NL) * _NL, _NL)]
                pos = jnp.sum(jnp.where(lane == (r % _NL), vec, zero))
                b_abs = (base + r) >> 1
                pltpu.make_async_copy(
                    table_hbm.at[pl.ds(pos, 1), pl.ds(0, D_SC),
                                 pl.ds((b_abs // BT) * BT, BT)],
                    slab_v.at[buf].at[pl.ds(k_, 1)],
                    sems[buf],
                ).start()

        def drain(w):
            buf = w % 2
            pltpu.make_async_copy(
                table_hbm.at[pl.ds(0, 1), pl.ds(0, D_SC), pl.ds(0, BT)],
                slab_v.at[buf].at[pl.ds(0, 1)],
                sems[buf],
            ).wait()

        def select(w):
            buf = w % 2
            for k_ in range(WAVE):
                r = w * WAVE + k_
                b_in = ((base + r) >> 1) % BT
                col = jnp.full((_NL,), b_in, dtype=jnp.int32)
                kvec = jnp.full((_NL,), k_, dtype=jnp.int32)
                starts = range(0, D_SC, _NL) if D_SC % _NL == 0 else (
                    [0, D_SC - _NL])
                pvec = jnp.full((_NL,), r & 1, dtype=jnp.int32)
                for d0 in starts:
                    dvec = lane + d0
                    val = plsc.load_gather(
                        slab_v.at[buf], [kvec, dvec, col])
                    plsc.store_scatter(rows_v, [pvec, dvec, col], val)

        fire(0)
        for w in range(N_WAVES):
            if w + 1 < N_WAVES:
                fire(w + 1)
            for k_ in range(WAVE - 1):
                drain(w)
            drain(w)
            select(w)
        pltpu.sync_copy(rows_v, out_hbm.at[:, :, pl.ds(wid * BT, BT)])

    return k(table_t, idx_flat)


B_BLK = 4096
CHUNK_L = 100
N_LSTEP = L_SENT // CHUNK_L
D_BLK = 8
N_DSTEP = D_TC // D_BLK
D_SKIP = D_SC // D_BLK


def _tc_body(idx_ref, tab_ref, out_ref):
    i = pl.program_id(1)

    @pl.when(i == 0)
    def _init():
        out_ref[...] = jnp.zeros_like(out_ref)

    lvals = idx_ref[...]
    for q in range(CHUNK_L):
        tab = tab_ref[q]
        l_cur = i * CHUNK_L + q
        for p in range(P):
            sel = lvals[p, :][None, :] == l_cur
            out_ref[p] = jnp.where(sel, tab, out_ref[p])


def _tc_gather(table_t, idx_t):
    return pl.pallas_call(
        _tc_body,
        grid=(N_DSTEP, N_LSTEP),
        in_specs=[
            pl.BlockSpec((P, B_BLK), lambda j, i: (0, 0)),
            pl.BlockSpec((CHUNK_L, D_BLK, B_BLK),
                         lambda j, i: (i, D_SKIP + j, 0)),
        ],
        out_specs=pl.BlockSpec((P, D_BLK, B_BLK), lambda j, i: (0, j, 0)),
        out_shape=jax.ShapeDtypeStruct((P, D_TC, B), jnp.float32),
    )(idx_t, table_t)


def kernel(sentence_matrix, entity_pos_index):
    table_t = jnp.transpose(sentence_matrix, (1, 2, 0))
    idx32 = entity_pos_index.astype(jnp.int32)
    sc_out = _sc_gather(table_t, idx32.reshape(ROWS))
    tc_out = _tc_gather(table_t, idx32.T)
    out_t = jnp.concatenate([sc_out, tc_out], axis=1)
    return jnp.transpose(out_t, (2, 0, 1))
